# Initial kernel scaffold; baseline (speedup 1.0000x reference)
import jax
import jax.numpy as jnp
from jax.experimental import pallas as pl


def kernel(x, edge_index, edge_attr, params):
    raise NotImplementedError("write your pallas kernel here")


# speedup vs baseline: 1.0127x; 1.0127x over previous
import functools

import jax
import jax.numpy as jnp
from jax import lax
from jax.experimental import pallas as pl
from jax.experimental.pallas import tpu as pltpu
from jax.experimental.pallas import tpu_sc as plsc

N_LAYERS = 3
D = 128
N = 10000
E = 320000

NC = 2
NS = 16
NW = NC * NS
EW = E // NW
C = 80
N_PAD = 10240
ROWS_PER_TILE = N_PAD // NS

@functools.cache
def _sc_mesh():
    return plsc.VectorSubcoreMesh(core_axis_name="c", subcore_axis_name="s")


def _cnt_body(src_hbm, zeros_hbm, ones_hbm, out_hbm, idx_v, ones_v, cnt_sh, sem):
    cid = lax.axis_index("c")
    sid = lax.axis_index("s")
    wid = sid * NC + cid
    pltpu.sync_copy(zeros_hbm.at[pl.ds(sid * ROWS_PER_TILE, ROWS_PER_TILE)],
                    cnt_sh.at[pl.ds(sid * ROWS_PER_TILE, ROWS_PER_TILE)])
    pltpu.sync_copy(ones_hbm, ones_v)
    plsc.subcore_barrier()

    def chunk(k, carry):
        base = wid * EW + k * C
        pltpu.sync_copy(src_hbm.at[pl.ds(base, C)], idx_v)
        pltpu.sync_copy(ones_v, cnt_sh.at[idx_v], add=True)
        return carry

    lax.fori_loop(0, EW // C, chunk, 0)
    plsc.subcore_barrier()
    pltpu.sync_copy(cnt_sh.at[pl.ds(sid * ROWS_PER_TILE, ROWS_PER_TILE)],
                    out_hbm.at[cid, pl.ds(sid * ROWS_PER_TILE, ROWS_PER_TILE)])


@functools.cache
def _cnt_sc_call():
    return pl.kernel(
        _cnt_body,
        out_type=jax.ShapeDtypeStruct((NC, N_PAD, D), jnp.float32),
        mesh=_sc_mesh(),
        scratch_types=[
            pltpu.VMEM((C,), jnp.int32),
            pltpu.VMEM((C, D), jnp.float32),
            pltpu.VMEM_SHARED((N_PAD, D), jnp.float32),
            pltpu.SemaphoreType.DMA,
        ],
    )


def _cnt_sc(src, zeros_n16, ones_c16):
    return _cnt_sc_call()(src, zeros_n16, ones_c16)


def _edge_body(w_hbm, dst_hbm, src_hbm, ta_hbm, tb_hbm, zeros_hbm,
               s_hbm, agg_hbm,
               idxd_v, idxs_v, buf_a, buf_b, buf_w, agg_sh, sem_a, sem_b):
    cid = lax.axis_index("c")
    sid = lax.axis_index("s")
    wid = sid * NC + cid
    pltpu.sync_copy(zeros_hbm.at[pl.ds(sid * ROWS_PER_TILE, ROWS_PER_TILE)],
                    agg_sh.at[pl.ds(sid * ROWS_PER_TILE, ROWS_PER_TILE)])
    plsc.subcore_barrier()

    def chunk(k, carry):
        base = wid * EW + k * C
        pltpu.sync_copy(dst_hbm.at[pl.ds(base, C)], idxd_v)
        pltpu.sync_copy(src_hbm.at[pl.ds(base, C)], idxs_v)
        ga = pltpu.async_copy(ta_hbm.at[idxd_v], buf_a, sem_a)
        gb = pltpu.async_copy(tb_hbm.at[idxs_v], buf_b, sem_b)
        pltpu.sync_copy(w_hbm.at[pl.ds(base, C)], buf_w)
        ga.wait()
        gb.wait()

        def row(r, rc):
            for j in range(D // 16):
                sl = pl.ds(j * 16, 16)
                wv = buf_w[r, sl]
                sig = 1.0 / (1.0 + jnp.exp(-wv))
                buf_w[r, sl] = sig * buf_a[r, sl]
                buf_b[r, sl] = buf_b[r, sl] + buf_a[r, pl.ds(D + j * 16, 16)]
            return rc

        lax.fori_loop(0, C, row, 0)
        pltpu.sync_copy(buf_w, agg_sh.at[idxs_v], add=True)
        pltpu.sync_copy(buf_b, s_hbm.at[pl.ds(base, C)])
        return carry

    lax.fori_loop(0, EW // C, chunk, 0)
    plsc.subcore_barrier()
    pltpu.sync_copy(agg_sh.at[pl.ds(sid * ROWS_PER_TILE, ROWS_PER_TILE)],
                    agg_hbm.at[cid, pl.ds(sid * ROWS_PER_TILE, ROWS_PER_TILE)])


@functools.cache
def _edge_sc_call():
    return pl.kernel(
        _edge_body,
        out_type=(
            jax.ShapeDtypeStruct((E, D), jnp.float32),
            jax.ShapeDtypeStruct((NC, N_PAD, D), jnp.float32),
        ),
        mesh=_sc_mesh(),
        scratch_types=[
            pltpu.VMEM((C,), jnp.int32),
            pltpu.VMEM((C,), jnp.int32),
            pltpu.VMEM((C, 2 * D), jnp.float32),
            pltpu.VMEM((C, D), jnp.float32),
            pltpu.VMEM((C, D), jnp.float32),
            pltpu.VMEM_SHARED((N_PAD, D), jnp.float32),
            pltpu.SemaphoreType.DMA,
            pltpu.SemaphoreType.DMA,
        ],
    )


def _edge_sc(w, dst, src, ta, tb, zeros_nd):
    return _edge_sc_call()(w, dst, src, ta, tb, zeros_nd)


def _silu_body(a_ref, o_ref):
    a = a_ref[...]
    o_ref[...] = a * jax.nn.sigmoid(a)


def _silu(a, block_rows):
    rows = a.shape[0]
    return pl.pallas_call(
        _silu_body,
        grid=(rows // block_rows,),
        in_specs=[pl.BlockSpec((block_rows, D), lambda i: (i, 0))],
        out_specs=pl.BlockSpec((block_rows, D), lambda i: (i, 0)),
        out_shape=jax.ShapeDtypeStruct((rows, D), jnp.float32),
    )(a)


def _n1_body(x_ref, w1, w2, w3, w4, b1, b2, b3, b4, x1_o, ta_o, tb_o):
    x = x_ref[...]
    x1_o[...] = jnp.dot(x, w1[...], preferred_element_type=jnp.float32) + b1[...]
    tb_o[...] = jnp.dot(x, w2[...], preferred_element_type=jnp.float32) + b2[...]
    ta_o[:, :D] = jnp.dot(x, w3[...], preferred_element_type=jnp.float32) + b3[...]
    ta_o[:, D:] = jnp.dot(x, w4[...], preferred_element_type=jnp.float32) + b4[...]


def _n1(x, p):
    return pl.pallas_call(
        _n1_body,
        out_shape=(
            jax.ShapeDtypeStruct((N, D), jnp.float32),
            jax.ShapeDtypeStruct((N, 2 * D), jnp.float32),
            jax.ShapeDtypeStruct((N, D), jnp.float32),
        ),
    )(x, p["W1"], p["W2"], p["W3"], p["W4"],
      p["b1"].reshape(1, D), p["b2"].reshape(1, D),
      p["b3"].reshape(1, D), p["b4"].reshape(1, D))


def _pass_a_body(w_ref, s_ref, we_ref, be_ref, h_ref, st_ref, acc_ref):
    i = pl.program_id(0)

    @pl.when(i == 0)
    def _():
        acc_ref[...] = jnp.zeros_like(acc_ref)

    e = w_ref[...] + s_ref[...]
    h = jnp.dot(e, we_ref[...], preferred_element_type=jnp.float32) + be_ref[...]
    h_ref[...] = h
    acc_ref[0:1, :] += jnp.sum(h, axis=0, keepdims=True)
    acc_ref[1:2, :] += jnp.sum(h * h, axis=0, keepdims=True)

    @pl.when(i == pl.num_programs(0) - 1)
    def _():
        st_ref[...] = acc_ref[...]


def _pass_a(w, s, p, block_rows):
    return pl.pallas_call(
        _pass_a_body,
        grid=(E // block_rows,),
        in_specs=[
            pl.BlockSpec((block_rows, D), lambda i: (i, 0)),
            pl.BlockSpec((block_rows, D), lambda i: (i, 0)),
            pl.BlockSpec((D, D), lambda i: (0, 0)),
            pl.BlockSpec((1, D), lambda i: (0, 0)),
        ],
        out_specs=(
            pl.BlockSpec((block_rows, D), lambda i: (i, 0)),
            pl.BlockSpec((8, D), lambda i: (0, 0)),
        ),
        out_shape=(
            jax.ShapeDtypeStruct((E, D), jnp.float32),
            jax.ShapeDtypeStruct((8, D), jnp.float32),
        ),
        scratch_shapes=[pltpu.VMEM((8, D), jnp.float32)],
    )(w, s, p["We"], p["be"].reshape(1, D))


def _pass_b_body(h_ref, w_ref, st_ref, g_ref, b_ref, o_ref):
    st = st_ref[...]
    mu = st[0:1, :] * (1.0 / E)
    var = st[1:2, :] * (1.0 / E) - mu * mu
    scale = lax.rsqrt(var + 1e-5) * g_ref[...]
    t = (h_ref[...] - mu) * scale + b_ref[...]
    o_ref[...] = w_ref[...] + t * jax.nn.sigmoid(t)


def _pass_b(h, w, stats, p, block_rows):
    return pl.pallas_call(
        _pass_b_body,
        grid=(E // block_rows,),
        in_specs=[
            pl.BlockSpec((block_rows, D), lambda i: (i, 0)),
            pl.BlockSpec((block_rows, D), lambda i: (i, 0)),
            pl.BlockSpec((8, D), lambda i: (0, 0)),
            pl.BlockSpec((1, D), lambda i: (0, 0)),
            pl.BlockSpec((1, D), lambda i: (0, 0)),
        ],
        out_specs=pl.BlockSpec((block_rows, D), lambda i: (i, 0)),
        out_shape=jax.ShapeDtypeStruct((E, D), jnp.float32),
    )(h, w, stats, p["e_gamma"].reshape(1, D), p["e_beta"].reshape(1, D))


def _n2_body(x0_ref, x1_ref, aggp_ref, cntp_ref, g_ref, b_ref, o_ref):
    a = aggp_ref[0, :N, :] + aggp_ref[1, :N, :]
    c = cntp_ref[0, :N, 0:1] + cntp_ref[1, :N, 0:1]
    t = x1_ref[...] + a / jnp.maximum(c, 1.0)
    mu = jnp.mean(t, axis=0, keepdims=True)
    var = jnp.mean(jnp.square(t - mu), axis=0, keepdims=True)
    bn = g_ref[...] * (t - mu) * lax.rsqrt(var + 1e-5) + b_ref[...]
    o_ref[...] = x0_ref[...] + bn * jax.nn.sigmoid(bn)


def _n2(x0, x1, aggp, cntp, p):
    return pl.pallas_call(
        _n2_body,
        out_shape=jax.ShapeDtypeStruct((N, D), jnp.float32),
    )(x0, x1, aggp, cntp, p["v_gamma"].reshape(1, D), p["v_beta"].reshape(1, D))


def kernel(x, edge_index, edge_attr, params):
    src = edge_index[0]
    dst = edge_index[1]

    x_cur = _silu(x, 2000)
    w_cur = _silu(edge_attr, 4000)

    zeros_nd = jnp.zeros((N_PAD, D), jnp.float32)
    zeros_n16 = jnp.zeros((N_PAD, D), jnp.float32)
    ones_c16 = jnp.ones((C, D), jnp.float32)
    cntp = _cnt_sc(src, zeros_n16, ones_c16)

    for p in params:
        x1, ta, tb = _n1(x_cur, p)
        s, aggp = _edge_sc(w_cur, dst, src, ta, tb, zeros_nd)
        h, stats = _pass_a(w_cur, s, p, 2000)
        x_cur = _n2(x_cur, x1, aggp, cntp, p)
        w_cur = _pass_b(h, w_cur, stats, p, 4000)

    return x_cur, w_cur


# speedup vs baseline: 3.0210x; 2.9831x over previous
import functools

import jax
import jax.numpy as jnp
from jax import lax
from jax.experimental import pallas as pl
from jax.experimental.pallas import tpu as pltpu
from jax.experimental.pallas import tpu_sc as plsc

N_LAYERS = 3
D = 128
N = 10000
E = 320000

NC = 2
NS = 16
NW = NC * NS
EW = E // NW
C = 80
N_PAD = 10240
ROWS_PER_TILE = N_PAD // NS

@functools.cache
def _sc_mesh():
    return plsc.VectorSubcoreMesh(core_axis_name="c", subcore_axis_name="s")


def _cnt_body(src_hbm, zeros_hbm, ones_hbm, out_hbm, idx_v, ones_v, cnt_sh, sem):
    cid = lax.axis_index("c")
    sid = lax.axis_index("s")
    wid = sid * NC + cid
    pltpu.sync_copy(zeros_hbm.at[pl.ds(sid * ROWS_PER_TILE, ROWS_PER_TILE)],
                    cnt_sh.at[pl.ds(sid * ROWS_PER_TILE, ROWS_PER_TILE)])
    pltpu.sync_copy(ones_hbm, ones_v)
    plsc.subcore_barrier()

    def chunk(k, carry):
        base = wid * EW + k * C
        pltpu.sync_copy(src_hbm.at[pl.ds(base, C)], idx_v)
        pltpu.sync_copy(ones_v, cnt_sh.at[idx_v], add=True)
        return carry

    lax.fori_loop(0, EW // C, chunk, 0)
    plsc.subcore_barrier()
    pltpu.sync_copy(cnt_sh.at[pl.ds(sid * ROWS_PER_TILE, ROWS_PER_TILE)],
                    out_hbm.at[cid, pl.ds(sid * ROWS_PER_TILE, ROWS_PER_TILE)])


@functools.cache
def _cnt_sc_call():
    return pl.kernel(
        _cnt_body,
        out_type=jax.ShapeDtypeStruct((NC, N_PAD, D), jnp.float32),
        mesh=_sc_mesh(),
        scratch_types=[
            pltpu.VMEM((C,), jnp.int32),
            pltpu.VMEM((C, D), jnp.float32),
            pltpu.VMEM_SHARED((N_PAD, D), jnp.float32),
            pltpu.SemaphoreType.DMA,
        ],
    )


def _cnt_sc(src, zeros_n16, ones_c16):
    return _cnt_sc_call()(src, zeros_n16, ones_c16)


NCH = EW // C


def _scat_body(w_hbm, dst_hbm, src_hbm, t3_hbm, zeros_hbm,
               agg_hbm,
               id0, id1, id2, is0, is1, is2, a0, a1, w0, w1, agg_sh,
               si0, si1, si2, sg0, sg1, ss0, ss1):
    cid = lax.axis_index("c")
    sid = lax.axis_index("s")
    wid = sid * NC + cid
    wbase = wid * EW
    pltpu.sync_copy(zeros_hbm.at[pl.ds(sid * ROWS_PER_TILE, ROWS_PER_TILE)],
                    agg_sh.at[pl.ds(sid * ROWS_PER_TILE, ROWS_PER_TILE)])
    plsc.subcore_barrier()

    idxd = (id0, id1, id2)
    idxs = (is0, is1, is2)
    abufs = (a0, a1)
    wbufs = (w0, w1)
    isems = (si0, si1, si2)
    gsems = (sg0, sg1)
    ssems = (ss0, ss1)

    def fire_idx(k, i):
        pltpu.async_copy(dst_hbm.at[pl.ds(wbase + k * C, C)], idxd[i], isems[i])
        pltpu.async_copy(src_hbm.at[pl.ds(wbase + k * C, C)], idxs[i], isems[i])

    def wait_idx(i):
        pltpu.make_async_copy(dst_hbm.at[pl.ds(wbase, C)], idxd[i], isems[i]).wait()
        pltpu.make_async_copy(src_hbm.at[pl.ds(wbase, C)], idxs[i], isems[i]).wait()

    def fire_g(k, b, i):
        pltpu.async_copy(t3_hbm.at[idxd[i]], abufs[b], gsems[b])
        pltpu.async_copy(w_hbm.at[pl.ds(wbase + k * C, C)], wbufs[b], gsems[b])

    def wait_g(b):
        pltpu.make_async_copy(t3_hbm.at[idxd[0]], abufs[b], gsems[b]).wait()
        pltpu.make_async_copy(w_hbm.at[pl.ds(wbase, C)], wbufs[b], gsems[b]).wait()

    def fire_out(b, i):
        pltpu.async_copy(wbufs[b], agg_sh.at[idxs[i]], ssems[b], add=True)

    def wait_out(b):
        pltpu.make_async_copy(wbufs[b], agg_sh.at[idxs[0]], ssems[b]).wait()

    def compute(b):
        a_c, w_c = abufs[b], wbufs[b]

        def row(r, rc):
            for j in range(D // 16):
                sl = pl.ds(j * 16, 16)
                wv = w_c[r, sl]
                sig = 1.0 / (1.0 + jnp.exp(-wv))
                w_c[r, sl] = sig * a_c[r, sl]
            return rc

        lax.fori_loop(0, C, row, 0, unroll=4)

    fire_idx(0, 0)
    fire_idx(1, 1)
    wait_idx(0)
    fire_g(0, 0, 0)

    def iter_k(k, c):
        def step(bpar, ipar):
            bn = 1 - bpar
            i_n = (ipar + 1) % 3
            i_f = (ipar + 2) % 3

            @pl.when(k > 0)
            def _():
                wait_out(bn)

            @pl.when(k + 1 < NCH)
            def _():
                wait_idx(i_n)
                fire_g(k + 1, bn, i_n)

            @pl.when(k + 2 < NCH)
            def _():
                fire_idx(k + 2, i_f)

            wait_g(bpar)
            compute(bpar)
            fire_out(bpar, ipar)

        for par in range(6):
            @pl.when(k % 6 == par)
            def _(par=par):
                step(par % 2, par % 3)

        return c

    lax.fori_loop(0, NCH, iter_k, 0)
    wait_out((NCH - 1) % 2)
    plsc.subcore_barrier()
    pltpu.sync_copy(agg_sh.at[pl.ds(sid * ROWS_PER_TILE, ROWS_PER_TILE)],
                    agg_hbm.at[cid, pl.ds(sid * ROWS_PER_TILE, ROWS_PER_TILE)])


@functools.cache
def _scat_sc_call():
    return pl.kernel(
        _scat_body,
        out_type=jax.ShapeDtypeStruct((NC, N_PAD, D), jnp.float32),
        mesh=_sc_mesh(),
        scratch_types=[
            pltpu.VMEM((C,), jnp.int32), pltpu.VMEM((C,), jnp.int32),
            pltpu.VMEM((C,), jnp.int32), pltpu.VMEM((C,), jnp.int32),
            pltpu.VMEM((C,), jnp.int32), pltpu.VMEM((C,), jnp.int32),
            pltpu.VMEM((C, D), jnp.float32), pltpu.VMEM((C, D), jnp.float32),
            pltpu.VMEM((C, D), jnp.float32), pltpu.VMEM((C, D), jnp.float32),
            pltpu.VMEM_SHARED((N_PAD, D), jnp.float32),
            pltpu.SemaphoreType.DMA, pltpu.SemaphoreType.DMA,
            pltpu.SemaphoreType.DMA, pltpu.SemaphoreType.DMA,
            pltpu.SemaphoreType.DMA, pltpu.SemaphoreType.DMA,
            pltpu.SemaphoreType.DMA,
        ],
    )


def _scat_sc(w, dst, src, t3, zeros_nd):
    return _scat_sc_call()(w, dst, src, t3, zeros_nd)


def _sgat_body(dst_hbm, src_hbm, t2_hbm, t4_hbm,
               s_hbm,
               id0, id1, id2, is0, is1, is2, a0, a1, b0, b1,
               si0, si1, si2, sg0, sg1, ss0, ss1):
    cid = lax.axis_index("c")
    sid = lax.axis_index("s")
    wid = sid * NC + cid
    wbase = wid * EW

    idxd = (id0, id1, id2)
    idxs = (is0, is1, is2)
    abufs = (a0, a1)
    bbufs = (b0, b1)
    isems = (si0, si1, si2)
    gsems = (sg0, sg1)
    ssems = (ss0, ss1)

    def fire_idx(k, i):
        pltpu.async_copy(dst_hbm.at[pl.ds(wbase + k * C, C)], idxd[i], isems[i])
        pltpu.async_copy(src_hbm.at[pl.ds(wbase + k * C, C)], idxs[i], isems[i])

    def wait_idx(i):
        pltpu.make_async_copy(dst_hbm.at[pl.ds(wbase, C)], idxd[i], isems[i]).wait()
        pltpu.make_async_copy(src_hbm.at[pl.ds(wbase, C)], idxs[i], isems[i]).wait()

    def fire_g(b, i):
        pltpu.async_copy(t4_hbm.at[idxd[i]], abufs[b], gsems[b])
        pltpu.async_copy(t2_hbm.at[idxs[i]], bbufs[b], gsems[b])

    def wait_g(b):
        pltpu.make_async_copy(t4_hbm.at[idxd[0]], abufs[b], gsems[b]).wait()
        pltpu.make_async_copy(t2_hbm.at[idxs[0]], bbufs[b], gsems[b]).wait()

    def fire_out(k, b):
        pltpu.async_copy(bbufs[b], s_hbm.at[pl.ds(wbase + k * C, C)], ssems[b])

    def wait_out(b):
        pltpu.make_async_copy(bbufs[b], s_hbm.at[pl.ds(wbase, C)], ssems[b]).wait()

    def compute(b):
        a_c, b_c = abufs[b], bbufs[b]

        def row(r, rc):
            for j in range(D // 16):
                sl = pl.ds(j * 16, 16)
                b_c[r, sl] = b_c[r, sl] + a_c[r, sl]
            return rc

        lax.fori_loop(0, C, row, 0, unroll=4)

    fire_idx(0, 0)
    fire_idx(1, 1)
    wait_idx(0)
    fire_g(0, 0)

    def iter_k(k, c):
        def step(bpar, ipar):
            bn = 1 - bpar
            i_n = (ipar + 1) % 3
            i_f = (ipar + 2) % 3

            @pl.when(k > 0)
            def _():
                wait_out(bn)

            @pl.when(k + 1 < NCH)
            def _():
                wait_idx(i_n)
                fire_g(bn, i_n)

            @pl.when(k + 2 < NCH)
            def _():
                fire_idx(k + 2, i_f)

            wait_g(bpar)
            compute(bpar)
            fire_out(k, bpar)

        for par in range(6):
            @pl.when(k % 6 == par)
            def _(par=par):
                step(par % 2, par % 3)

        return c

    lax.fori_loop(0, NCH, iter_k, 0)
    wait_out((NCH - 1) % 2)


@functools.cache
def _sgat_sc_call():
    return pl.kernel(
        _sgat_body,
        out_type=jax.ShapeDtypeStruct((E, D), jnp.float32),
        mesh=_sc_mesh(),
        scratch_types=[
            pltpu.VMEM((C,), jnp.int32), pltpu.VMEM((C,), jnp.int32),
            pltpu.VMEM((C,), jnp.int32), pltpu.VMEM((C,), jnp.int32),
            pltpu.VMEM((C,), jnp.int32), pltpu.VMEM((C,), jnp.int32),
            pltpu.VMEM((C, D), jnp.float32), pltpu.VMEM((C, D), jnp.float32),
            pltpu.VMEM((C, D), jnp.float32), pltpu.VMEM((C, D), jnp.float32),
            pltpu.SemaphoreType.DMA, pltpu.SemaphoreType.DMA,
            pltpu.SemaphoreType.DMA, pltpu.SemaphoreType.DMA,
            pltpu.SemaphoreType.DMA, pltpu.SemaphoreType.DMA,
            pltpu.SemaphoreType.DMA,
        ],
    )


def _sgat_sc(dst, src, t2, t4):
    return _sgat_sc_call()(dst, src, t2, t4)


def _silu_body(a_ref, o_ref):
    a = a_ref[...]
    o_ref[...] = a * jax.nn.sigmoid(a)


def _silu(a, block_rows):
    rows = a.shape[0]
    return pl.pallas_call(
        _silu_body,
        grid=(rows // block_rows,),
        in_specs=[pl.BlockSpec((block_rows, D), lambda i: (i, 0))],
        out_specs=pl.BlockSpec((block_rows, D), lambda i: (i, 0)),
        out_shape=jax.ShapeDtypeStruct((rows, D), jnp.float32),
    )(a)


def _n1_body(x_ref, w1, w2, w3, w4, b1, b2, b3, b4, x1_o, x2_o, x3_o, x4_o):
    x = x_ref[...]
    x1_o[...] = jnp.dot(x, w1[...], preferred_element_type=jnp.float32) + b1[...]
    x2_o[...] = jnp.dot(x, w2[...], preferred_element_type=jnp.float32) + b2[...]
    x3_o[...] = jnp.dot(x, w3[...], preferred_element_type=jnp.float32) + b3[...]
    x4_o[...] = jnp.dot(x, w4[...], preferred_element_type=jnp.float32) + b4[...]


def _n1(x, p):
    return pl.pallas_call(
        _n1_body,
        out_shape=tuple(jax.ShapeDtypeStruct((N, D), jnp.float32) for _ in range(4)),
    )(x, p["W1"], p["W2"], p["W3"], p["W4"],
      p["b1"].reshape(1, D), p["b2"].reshape(1, D),
      p["b3"].reshape(1, D), p["b4"].reshape(1, D))


def _pass_a_body(w_ref, s_ref, we_ref, be_ref, h_ref, st_ref, acc_ref):
    i = pl.program_id(0)

    @pl.when(i == 0)
    def _():
        acc_ref[...] = jnp.zeros_like(acc_ref)

    e = w_ref[...] + s_ref[...]
    h = jnp.dot(e, we_ref[...], preferred_element_type=jnp.float32) + be_ref[...]
    h_ref[...] = h
    acc_ref[0:1, :] += jnp.sum(h, axis=0, keepdims=True)
    acc_ref[1:2, :] += jnp.sum(h * h, axis=0, keepdims=True)

    @pl.when(i == pl.num_programs(0) - 1)
    def _():
        st_ref[...] = acc_ref[...]


def _pass_a(w, s, p, block_rows):
    return pl.pallas_call(
        _pass_a_body,
        grid=(E // block_rows,),
        in_specs=[
            pl.BlockSpec((block_rows, D), lambda i: (i, 0)),
            pl.BlockSpec((block_rows, D), lambda i: (i, 0)),
            pl.BlockSpec((D, D), lambda i: (0, 0)),
            pl.BlockSpec((1, D), lambda i: (0, 0)),
        ],
        out_specs=(
            pl.BlockSpec((block_rows, D), lambda i: (i, 0)),
            pl.BlockSpec((8, D), lambda i: (0, 0)),
        ),
        out_shape=(
            jax.ShapeDtypeStruct((E, D), jnp.float32),
            jax.ShapeDtypeStruct((8, D), jnp.float32),
        ),
        scratch_shapes=[pltpu.VMEM((8, D), jnp.float32)],
    )(w, s, p["We"], p["be"].reshape(1, D))


def _pass_b_body(h_ref, w_ref, st_ref, g_ref, b_ref, o_ref):
    st = st_ref[...]
    mu = st[0:1, :] * (1.0 / E)
    var = st[1:2, :] * (1.0 / E) - mu * mu
    scale = lax.rsqrt(var + 1e-5) * g_ref[...]
    t = (h_ref[...] - mu) * scale + b_ref[...]
    o_ref[...] = w_ref[...] + t * jax.nn.sigmoid(t)


def _pass_b(h, w, stats, p, block_rows):
    return pl.pallas_call(
        _pass_b_body,
        grid=(E // block_rows,),
        in_specs=[
            pl.BlockSpec((block_rows, D), lambda i: (i, 0)),
            pl.BlockSpec((block_rows, D), lambda i: (i, 0)),
            pl.BlockSpec((8, D), lambda i: (0, 0)),
            pl.BlockSpec((1, D), lambda i: (0, 0)),
            pl.BlockSpec((1, D), lambda i: (0, 0)),
        ],
        out_specs=pl.BlockSpec((block_rows, D), lambda i: (i, 0)),
        out_shape=jax.ShapeDtypeStruct((E, D), jnp.float32),
    )(h, w, stats, p["e_gamma"].reshape(1, D), p["e_beta"].reshape(1, D))


def _n2_body(x0_ref, x1_ref, aggp_ref, cntp_ref, g_ref, b_ref, o_ref):
    a = aggp_ref[0, :N, :] + aggp_ref[1, :N, :]
    c = cntp_ref[0, :N, 0:1] + cntp_ref[1, :N, 0:1]
    t = x1_ref[...] + a / jnp.maximum(c, 1.0)
    mu = jnp.mean(t, axis=0, keepdims=True)
    var = jnp.mean(jnp.square(t - mu), axis=0, keepdims=True)
    bn = g_ref[...] * (t - mu) * lax.rsqrt(var + 1e-5) + b_ref[...]
    o_ref[...] = x0_ref[...] + bn * jax.nn.sigmoid(bn)


def _n2(x0, x1, aggp, cntp, p):
    return pl.pallas_call(
        _n2_body,
        out_shape=jax.ShapeDtypeStruct((N, D), jnp.float32),
    )(x0, x1, aggp, cntp, p["v_gamma"].reshape(1, D), p["v_beta"].reshape(1, D))


def kernel(x, edge_index, edge_attr, params):
    src = edge_index[0]
    dst = edge_index[1]

    x_cur = _silu(x, 2000)
    w_cur = _silu(edge_attr, 4000)

    zeros_nd = jnp.zeros((N_PAD, D), jnp.float32)
    zeros_n16 = jnp.zeros((N_PAD, D), jnp.float32)
    ones_c16 = jnp.ones((C, D), jnp.float32)
    cntp = _cnt_sc(src, zeros_n16, ones_c16)

    for p in params:
        x1, x2, x3, x4 = _n1(x_cur, p)
        aggp = _scat_sc(w_cur, dst, src, x3, zeros_nd)
        s = _sgat_sc(dst, src, x2, x4)
        h, stats = _pass_a(w_cur, s, p, 2000)
        x_cur = _n2(x_cur, x1, aggp, cntp, p)
        w_cur = _pass_b(h, w_cur, stats, p, 4000)

    return x_cur, w_cur


# speedup vs baseline: 3.0307x; 1.0032x over previous
import functools

import jax
import jax.numpy as jnp
from jax import lax
from jax.experimental import pallas as pl
from jax.experimental.pallas import tpu as pltpu
from jax.experimental.pallas import tpu_sc as plsc

N_LAYERS = 3
D = 128
N = 10000
E = 320000

NC = 2
NS = 16
NW = NC * NS
EW = E // NW
C = 80
N_PAD = 10240
ROWS_PER_TILE = N_PAD // NS

@functools.cache
def _sc_mesh():
    return plsc.VectorSubcoreMesh(core_axis_name="c", subcore_axis_name="s")


def _cnt_body(src_hbm, zeros_hbm, ones_hbm, out_hbm, idx_v, ones_v, cnt_sh, sem):
    cid = lax.axis_index("c")
    sid = lax.axis_index("s")
    wid = sid * NC + cid
    pltpu.sync_copy(zeros_hbm.at[pl.ds(sid * ROWS_PER_TILE, ROWS_PER_TILE)],
                    cnt_sh.at[pl.ds(sid * ROWS_PER_TILE, ROWS_PER_TILE)])
    pltpu.sync_copy(ones_hbm, ones_v)
    plsc.subcore_barrier()

    def chunk(k, carry):
        base = wid * EW + k * C
        pltpu.sync_copy(src_hbm.at[pl.ds(base, C)], idx_v)
        pltpu.sync_copy(ones_v, cnt_sh.at[idx_v], add=True)
        return carry

    lax.fori_loop(0, EW // C, chunk, 0)
    plsc.subcore_barrier()
    pltpu.sync_copy(cnt_sh.at[pl.ds(sid * ROWS_PER_TILE, ROWS_PER_TILE)],
                    out_hbm.at[cid, pl.ds(sid * ROWS_PER_TILE, ROWS_PER_TILE)])


@functools.cache
def _cnt_sc_call():
    return pl.kernel(
        _cnt_body,
        out_type=jax.ShapeDtypeStruct((NC, N_PAD, D), jnp.float32),
        mesh=_sc_mesh(),
        scratch_types=[
            pltpu.VMEM((C,), jnp.int32),
            pltpu.VMEM((C, D), jnp.float32),
            pltpu.VMEM_SHARED((N_PAD, D), jnp.float32),
            pltpu.SemaphoreType.DMA,
        ],
    )


def _cnt_sc(src, zeros_n16, ones_c16):
    return _cnt_sc_call()(src, zeros_n16, ones_c16)


CS = 40
NCHS = EW // CS
NCH = EW // C


def _scat_body(w_hbm, dst_hbm, src_hbm, t3_hbm, zeros_hbm,
               agg_hbm,
               id0, id1, id2, id3, id4, id5,
               is0, is1, is2, is3, is4, is5,
               a0, a1, w0, w1, w2, agg_sh,
               si0, si1, si2, si3, si4, si5,
               sg0, sg1, ss0, ss1, ss2):
    cid = lax.axis_index("c")
    sid = lax.axis_index("s")
    wid = sid * NC + cid
    wbase = wid * EW
    pltpu.sync_copy(zeros_hbm.at[pl.ds(sid * ROWS_PER_TILE, ROWS_PER_TILE)],
                    agg_sh.at[pl.ds(sid * ROWS_PER_TILE, ROWS_PER_TILE)])
    plsc.subcore_barrier()

    idxd = (id0, id1, id2, id3, id4, id5)
    idxs = (is0, is1, is2, is3, is4, is5)
    abufs = (a0, a1)
    wbufs = (w0, w1, w2)
    isems = (si0, si1, si2, si3, si4, si5)
    gsems = (sg0, sg1)
    ssems = (ss0, ss1, ss2)

    def fire_idx(k, i):
        pltpu.async_copy(dst_hbm.at[pl.ds(wbase + k * CS, CS)], idxd[i], isems[i])
        pltpu.async_copy(src_hbm.at[pl.ds(wbase + k * CS, CS)], idxs[i], isems[i])

    def wait_idx(i):
        pltpu.make_async_copy(dst_hbm.at[pl.ds(wbase, CS)], idxd[i], isems[i]).wait()
        pltpu.make_async_copy(src_hbm.at[pl.ds(wbase, CS)], idxs[i], isems[i]).wait()

    def fire_g(k, ba, bw, i):
        pltpu.async_copy(t3_hbm.at[idxd[i]], abufs[ba], gsems[ba])
        pltpu.async_copy(w_hbm.at[pl.ds(wbase + k * CS, CS)], wbufs[bw], gsems[ba])

    def wait_g(ba, bw):
        pltpu.make_async_copy(t3_hbm.at[idxd[0]], abufs[ba], gsems[ba]).wait()
        pltpu.make_async_copy(w_hbm.at[pl.ds(wbase, CS)], wbufs[bw], gsems[ba]).wait()

    def fire_out(bw, i):
        pltpu.async_copy(wbufs[bw], agg_sh.at[idxs[i]], ssems[bw], add=True)

    def wait_out(bw):
        pltpu.make_async_copy(wbufs[bw], agg_sh.at[idxs[0]], ssems[bw]).wait()

    def compute(ba, bw):
        a_c, w_c = abufs[ba], wbufs[bw]

        def row(r, rc):
            for j in range(D // 16):
                sl = pl.ds(j * 16, 16)
                wv = w_c[r, sl]
                sig = 1.0 / (1.0 + jnp.exp(-wv))
                w_c[r, sl] = sig * a_c[r, sl]
            return rc

        lax.fori_loop(0, CS, row, 0, unroll=4)

    fire_idx(0, 0)
    fire_idx(1, 1)
    wait_idx(0)
    fire_g(0, 0, 0, 0)

    def iter_k(k, c):
        def step(par):
            ba, bw = par % 2, par % 3
            ban, bwn = (par + 1) % 2, (par + 1) % 3
            i_n, i_f = (par + 1) % 6, (par + 2) % 6

            @pl.when(k > 1)
            def _():
                wait_out((par + 1) % 3)

            @pl.when(k + 1 < NCHS)
            def _():
                wait_idx(i_n)
                fire_g(k + 1, ban, bwn, i_n)

            @pl.when(k + 2 < NCHS)
            def _():
                fire_idx(k + 2, i_f)

            wait_g(ba, bw)
            compute(ba, bw)
            fire_out(bw, par)

        for par in range(6):
            @pl.when(k % 6 == par)
            def _(par=par):
                step(par)

        return c

    lax.fori_loop(0, NCHS, iter_k, 0)
    wait_out((NCHS - 2) % 3)
    wait_out((NCHS - 1) % 3)
    plsc.subcore_barrier()
    pltpu.sync_copy(agg_sh.at[pl.ds(sid * ROWS_PER_TILE, ROWS_PER_TILE)],
                    agg_hbm.at[cid, pl.ds(sid * ROWS_PER_TILE, ROWS_PER_TILE)])


@functools.cache
def _scat_sc_call():
    return pl.kernel(
        _scat_body,
        out_type=jax.ShapeDtypeStruct((NC, N_PAD, D), jnp.float32),
        mesh=_sc_mesh(),
        scratch_types=(
            [pltpu.VMEM((CS,), jnp.int32)] * 12
            + [pltpu.VMEM((CS, D), jnp.float32)] * 5
            + [pltpu.VMEM_SHARED((N_PAD, D), jnp.float32)]
            + [pltpu.SemaphoreType.DMA] * 11
        ),
    )


def _scat_sc(w, dst, src, t3, zeros_nd):
    return _scat_sc_call()(w, dst, src, t3, zeros_nd)


def _sgat_body(dst_hbm, src_hbm, t2_hbm, t4_hbm,
               s_hbm,
               id0, id1, id2, id3, id4, id5,
               is0, is1, is2, is3, is4, is5,
               a0, a1, b0, b1, b2,
               si0, si1, si2, si3, si4, si5,
               sg0, sg1, ss0, ss1, ss2):
    cid = lax.axis_index("c")
    sid = lax.axis_index("s")
    wid = sid * NC + cid
    wbase = wid * EW

    idxd = (id0, id1, id2, id3, id4, id5)
    idxs = (is0, is1, is2, is3, is4, is5)
    abufs = (a0, a1)
    bbufs = (b0, b1, b2)
    isems = (si0, si1, si2, si3, si4, si5)
    gsems = (sg0, sg1)
    ssems = (ss0, ss1, ss2)

    def fire_idx(k, i):
        pltpu.async_copy(dst_hbm.at[pl.ds(wbase + k * C, C)], idxd[i], isems[i])
        pltpu.async_copy(src_hbm.at[pl.ds(wbase + k * C, C)], idxs[i], isems[i])

    def wait_idx(i):
        pltpu.make_async_copy(dst_hbm.at[pl.ds(wbase, C)], idxd[i], isems[i]).wait()
        pltpu.make_async_copy(src_hbm.at[pl.ds(wbase, C)], idxs[i], isems[i]).wait()

    def fire_g(ba, bb, i):
        pltpu.async_copy(t4_hbm.at[idxd[i]], abufs[ba], gsems[ba])
        pltpu.async_copy(t2_hbm.at[idxs[i]], bbufs[bb], gsems[ba])

    def wait_g(ba, bb):
        pltpu.make_async_copy(t4_hbm.at[idxd[0]], abufs[ba], gsems[ba]).wait()
        pltpu.make_async_copy(t2_hbm.at[idxs[0]], bbufs[bb], gsems[ba]).wait()

    def fire_out(k, bb):
        pltpu.async_copy(bbufs[bb], s_hbm.at[pl.ds(wbase + k * C, C)], ssems[bb])

    def wait_out(bb):
        pltpu.make_async_copy(bbufs[bb], s_hbm.at[pl.ds(wbase, C)], ssems[bb]).wait()

    def compute(ba, bb):
        a_c, b_c = abufs[ba], bbufs[bb]

        def row(r, rc):
            for j in range(D // 16):
                sl = pl.ds(j * 16, 16)
                b_c[r, sl] = b_c[r, sl] + a_c[r, sl]
            return rc

        lax.fori_loop(0, C, row, 0, unroll=4)

    fire_idx(0, 0)
    fire_idx(1, 1)
    wait_idx(0)
    fire_g(0, 0, 0)

    def iter_k(k, c):
        def step(par):
            ba, bb = par % 2, par % 3
            ban, bbn = (par + 1) % 2, (par + 1) % 3
            i_n, i_f = (par + 1) % 6, (par + 2) % 6

            @pl.when(k > 1)
            def _():
                wait_out((par + 1) % 3)

            @pl.when(k + 1 < NCH)
            def _():
                wait_idx(i_n)
                fire_g(ban, bbn, i_n)

            @pl.when(k + 2 < NCH)
            def _():
                fire_idx(k + 2, i_f)

            wait_g(ba, bb)
            compute(ba, bb)
            fire_out(k, bb)

        for par in range(6):
            @pl.when(k % 6 == par)
            def _(par=par):
                step(par)

        return c

    lax.fori_loop(0, NCH, iter_k, 0)
    wait_out((NCH - 2) % 3)
    wait_out((NCH - 1) % 3)


@functools.cache
def _sgat_sc_call():
    return pl.kernel(
        _sgat_body,
        out_type=jax.ShapeDtypeStruct((E, D), jnp.float32),
        mesh=_sc_mesh(),
        scratch_types=(
            [pltpu.VMEM((C,), jnp.int32)] * 12
            + [pltpu.VMEM((C, D), jnp.float32)] * 5
            + [pltpu.SemaphoreType.DMA] * 11
        ),
    )


def _sgat_sc(dst, src, t2, t4):
    return _sgat_sc_call()(dst, src, t2, t4)


def _silu_body(a_ref, o_ref):
    a = a_ref[...]
    o_ref[...] = a * jax.nn.sigmoid(a)


def _silu(a, block_rows):
    rows = a.shape[0]
    return pl.pallas_call(
        _silu_body,
        grid=(rows // block_rows,),
        in_specs=[pl.BlockSpec((block_rows, D), lambda i: (i, 0))],
        out_specs=pl.BlockSpec((block_rows, D), lambda i: (i, 0)),
        out_shape=jax.ShapeDtypeStruct((rows, D), jnp.float32),
    )(a)


def _n1_body(x_ref, w1, w2, w3, w4, b1, b2, b3, b4, x1_o, x2_o, x3_o, x4_o):
    x = x_ref[...]
    x1_o[...] = jnp.dot(x, w1[...], preferred_element_type=jnp.float32) + b1[...]
    x2_o[...] = jnp.dot(x, w2[...], preferred_element_type=jnp.float32) + b2[...]
    x3_o[...] = jnp.dot(x, w3[...], preferred_element_type=jnp.float32) + b3[...]
    x4_o[...] = jnp.dot(x, w4[...], preferred_element_type=jnp.float32) + b4[...]


def _n1(x, p):
    return pl.pallas_call(
        _n1_body,
        out_shape=tuple(jax.ShapeDtypeStruct((N, D), jnp.float32) for _ in range(4)),
    )(x, p["W1"], p["W2"], p["W3"], p["W4"],
      p["b1"].reshape(1, D), p["b2"].reshape(1, D),
      p["b3"].reshape(1, D), p["b4"].reshape(1, D))


def _pass_a_body(w_ref, s_ref, we_ref, be_ref, h_ref, st_ref, acc_ref):
    i = pl.program_id(0)

    @pl.when(i == 0)
    def _():
        acc_ref[...] = jnp.zeros_like(acc_ref)

    e = w_ref[...] + s_ref[...]
    h = jnp.dot(e, we_ref[...], preferred_element_type=jnp.float32) + be_ref[...]
    h_ref[...] = h
    acc_ref[0:1, :] += jnp.sum(h, axis=0, keepdims=True)
    acc_ref[1:2, :] += jnp.sum(h * h, axis=0, keepdims=True)

    @pl.when(i == pl.num_programs(0) - 1)
    def _():
        st_ref[...] = acc_ref[...]


def _pass_a(w, s, p, block_rows):
    return pl.pallas_call(
        _pass_a_body,
        grid=(E // block_rows,),
        in_specs=[
            pl.BlockSpec((block_rows, D), lambda i: (i, 0)),
            pl.BlockSpec((block_rows, D), lambda i: (i, 0)),
            pl.BlockSpec((D, D), lambda i: (0, 0)),
            pl.BlockSpec((1, D), lambda i: (0, 0)),
        ],
        out_specs=(
            pl.BlockSpec((block_rows, D), lambda i: (i, 0)),
            pl.BlockSpec((8, D), lambda i: (0, 0)),
        ),
        out_shape=(
            jax.ShapeDtypeStruct((E, D), jnp.float32),
            jax.ShapeDtypeStruct((8, D), jnp.float32),
        ),
        scratch_shapes=[pltpu.VMEM((8, D), jnp.float32)],
    )(w, s, p["We"], p["be"].reshape(1, D))


def _pass_b_body(h_ref, w_ref, st_ref, g_ref, b_ref, o_ref):
    st = st_ref[...]
    mu = st[0:1, :] * (1.0 / E)
    var = st[1:2, :] * (1.0 / E) - mu * mu
    scale = lax.rsqrt(var + 1e-5) * g_ref[...]
    t = (h_ref[...] - mu) * scale + b_ref[...]
    o_ref[...] = w_ref[...] + t * jax.nn.sigmoid(t)


def _pass_b(h, w, stats, p, block_rows):
    return pl.pallas_call(
        _pass_b_body,
        grid=(E // block_rows,),
        in_specs=[
            pl.BlockSpec((block_rows, D), lambda i: (i, 0)),
            pl.BlockSpec((block_rows, D), lambda i: (i, 0)),
            pl.BlockSpec((8, D), lambda i: (0, 0)),
            pl.BlockSpec((1, D), lambda i: (0, 0)),
            pl.BlockSpec((1, D), lambda i: (0, 0)),
        ],
        out_specs=pl.BlockSpec((block_rows, D), lambda i: (i, 0)),
        out_shape=jax.ShapeDtypeStruct((E, D), jnp.float32),
    )(h, w, stats, p["e_gamma"].reshape(1, D), p["e_beta"].reshape(1, D))


def _n2_body(x0_ref, x1_ref, aggp_ref, cntp_ref, g_ref, b_ref, o_ref):
    a = aggp_ref[0, :N, :] + aggp_ref[1, :N, :]
    c = cntp_ref[0, :N, 0:1] + cntp_ref[1, :N, 0:1]
    t = x1_ref[...] + a / jnp.maximum(c, 1.0)
    mu = jnp.mean(t, axis=0, keepdims=True)
    var = jnp.mean(jnp.square(t - mu), axis=0, keepdims=True)
    bn = g_ref[...] * (t - mu) * lax.rsqrt(var + 1e-5) + b_ref[...]
    o_ref[...] = x0_ref[...] + bn * jax.nn.sigmoid(bn)


def _n2(x0, x1, aggp, cntp, p):
    return pl.pallas_call(
        _n2_body,
        out_shape=jax.ShapeDtypeStruct((N, D), jnp.float32),
    )(x0, x1, aggp, cntp, p["v_gamma"].reshape(1, D), p["v_beta"].reshape(1, D))


def kernel(x, edge_index, edge_attr, params):
    src = edge_index[0]
    dst = edge_index[1]

    x_cur = _silu(x, 2000)
    w_cur = _silu(edge_attr, 4000)

    zeros_nd = jnp.zeros((N_PAD, D), jnp.float32)
    zeros_n16 = jnp.zeros((N_PAD, D), jnp.float32)
    ones_c16 = jnp.ones((C, D), jnp.float32)
    cntp = _cnt_sc(src, zeros_n16, ones_c16)

    for p in params:
        x1, x2, x3, x4 = _n1(x_cur, p)
        aggp = _scat_sc(w_cur, dst, src, x3, zeros_nd)
        s = _sgat_sc(dst, src, x2, x4)
        h, stats = _pass_a(w_cur, s, p, 2000)
        x_cur = _n2(x_cur, x1, aggp, cntp, p)
        w_cur = _pass_b(h, w_cur, stats, p, 4000)

    return x_cur, w_cur
